# baseline (speedup 1.0000x reference)
import functools

import jax
import jax.numpy as jnp
from jax import lax
from jax.experimental import pallas as pl
from jax.experimental.pallas import tpu as pltpu
from jax.experimental.pallas import tpu_sc as plsc

N = 10000
E = 320000
DX = 128
DE = 16
P1 = 64
P2 = 32
B = 64
EPS = 1e-5

NC = 2
NS = 16
NW = NC * NS
PER_W = E // NW
NSUP = 5
SUP = PER_W // NSUP
CH = 80
NCH = SUP // CH
ROWS_PER_TILE = N // NS


def _leaky(v):
    return jnp.where(v >= 0, v, 0.01 * v)


_SC_PARAMS = pltpu.CompilerParams(use_tc_tiling_on_sc=False)


def _prep_body(x_ref, w_ref, b_ref, o_ref):
    o_ref[...] = _leaky(
        jnp.dot(x_ref[...], w_ref[...], preferred_element_type=jnp.float32)
        + b_ref[...])


def _tc_prep(x, w0eff, b0):
    return pl.pallas_call(
        _prep_body,
        out_shape=jax.ShapeDtypeStruct((N, P2), jnp.float32),
    )(x, w0eff, b0)


MSG_T = 512


def _msg_body(ea_ref, u_ref, w1_ref, b1_ref, w2_ref, b2_ref, o_ref):
    ew1 = _leaky(
        jnp.dot(ea_ref[...], w1_ref[...], preferred_element_type=jnp.float32)
        + b1_ref[...])
    wt = jnp.dot(ew1, w2_ref[...], preferred_element_type=jnp.float32) + b2_ref[...]
    u = u_ref[...]
    acc = u[:, 0:1] * wt[:, 0:P2]
    for i in range(1, P2):
        acc += u[:, i:i + 1] * wt[:, i * P2:(i + 1) * P2]
    o_ref[...] = acc


def _tc_msg(ea, u, w1, b1, w2, b2):
    grid = (E // MSG_T,)
    return pl.pallas_call(
        _msg_body,
        grid=grid,
        in_specs=[
            pl.BlockSpec((MSG_T, DE), lambda i: (i, 0)),
            pl.BlockSpec((MSG_T, P2), lambda i: (i, 0)),
            pl.BlockSpec((DE, P1), lambda i: (0, 0)),
            pl.BlockSpec((1, P1), lambda i: (0, 0)),
            pl.BlockSpec((P1, P2 * P2), lambda i: (0, 0)),
            pl.BlockSpec((1, P2 * P2), lambda i: (0, 0)),
        ],
        out_specs=pl.BlockSpec((MSG_T, P2), lambda i: (i, 0)),
        out_shape=jax.ShapeDtypeStruct((E, P2), jnp.float32),
        compiler_params=pltpu.CompilerParams(
            dimension_semantics=("arbitrary",)),
    )(ea, u, w1, b1, w2, b2)


def _update_body(h_ref, ag_ref, ct_ref, wr_ref, bc_ref, wih_ref, whh_ref,
                 bih_ref, bhh_ref, o_ref):
    h = h_ref[...]
    aggr = (ag_ref[0] + ag_ref[1]) / jnp.maximum(ct_ref[0] + ct_ref[1], 1.0)
    m = _leaky(
        jnp.dot(h, wr_ref[...], preferred_element_type=jnp.float32)
        + aggr + bc_ref[...])
    gi = jnp.dot(m, wih_ref[...], preferred_element_type=jnp.float32) + bih_ref[...]
    gh = jnp.dot(h, whh_ref[...], preferred_element_type=jnp.float32) + bhh_ref[...]
    r = jax.nn.sigmoid(gi[:, 0:P2] + gh[:, 0:P2])
    z = jax.nn.sigmoid(gi[:, P2:2 * P2] + gh[:, P2:2 * P2])
    n = jnp.tanh(gi[:, 2 * P2:3 * P2] + r * gh[:, 2 * P2:3 * P2])
    o_ref[...] = (1.0 - z) * n + z * h


def _tc_update(h, aggr2, cnt2, wroot, bconv, wih, whh, bih, bhh):
    return pl.pallas_call(
        _update_body,
        out_shape=jax.ShapeDtypeStruct((N, P2), jnp.float32),
    )(h, aggr2, cnt2, wroot, bconv, wih, whh, bih, bhh)


def _s2s_body(x_ref, b_ref, lwih_ref, lwhh_ref, lbih_ref, lbhh_ref,
              w1_ref, b1_ref, w2_ref, b2_ref, wf_ref, bf_ref, y_ref):
    xo = x_ref[...]
    mask = b_ref[...] == lax.broadcasted_iota(jnp.int32, (N, B), 1)
    q_star = jnp.zeros((B, 2 * P2), jnp.float32)
    hl = jnp.zeros((B, P2), jnp.float32)
    cl = jnp.zeros((B, P2), jnp.float32)
    for _ in range(3):
        gates = (jnp.dot(q_star, lwih_ref[...], preferred_element_type=jnp.float32)
                 + lbih_ref[...]
                 + jnp.dot(hl, lwhh_ref[...], preferred_element_type=jnp.float32)
                 + lbhh_ref[...])
        i_ = jax.nn.sigmoid(gates[:, 0:P2])
        f_ = jax.nn.sigmoid(gates[:, P2:2 * P2])
        g_ = jnp.tanh(gates[:, 2 * P2:3 * P2])
        o_ = jax.nn.sigmoid(gates[:, 3 * P2:4 * P2])
        cl = f_ * cl + i_ * g_
        hl = o_ * jnp.tanh(cl)
        em = lax.dot_general(xo, hl, (((1,), (1,)), ((), ())),
                             preferred_element_type=jnp.float32)
        e = jnp.sum(jnp.where(mask, em, 0.0), axis=1, keepdims=True)
        emax = jnp.max(jnp.where(mask, e, -1e30), axis=0, keepdims=True)
        esel = jnp.sum(jnp.where(mask, emax, 0.0), axis=1, keepdims=True)
        a_un = jnp.exp(e - esel)
        masked_a = jnp.where(mask, a_un, 0.0)
        asum = jnp.maximum(jnp.sum(masked_a, axis=0, keepdims=True), 1e-30)
        wa = masked_a / asum
        rvec = lax.dot_general(wa, xo, (((0,), (0,)), ((), ())),
                               preferred_element_type=jnp.float32)
        q_star = jnp.concatenate([hl, rvec], axis=1)
    y = _leaky(jnp.dot(q_star, w1_ref[...], preferred_element_type=jnp.float32)
               + b1_ref[...])
    y = _leaky(jnp.dot(y, w2_ref[...], preferred_element_type=jnp.float32)
               + b2_ref[...])
    y_ref[...] = jnp.dot(y, wf_ref[...], preferred_element_type=jnp.float32) \
        + bf_ref[...]


def _tc_s2s(xo, batch2, lwih, lwhh, lbih, lbhh, w1, b1, w2, b2, wf, bf):
    return pl.pallas_call(
        _s2s_body,
        out_shape=jax.ShapeDtypeStruct((B, 1), jnp.float32),
    )(xo, batch2, lwih, lwhh, lbih, lbhh, w1, b1, w2, b2, wf, bf)


def _sc_gather(table, src):
    mesh = plsc.VectorSubcoreMesh(core_axis_name="c", subcore_axis_name="s")

    @functools.partial(
        pl.kernel,
        out_type=jax.ShapeDtypeStruct((E, P2), jnp.float32),
        mesh=mesh,
        compiler_params=_SC_PARAMS,
        scratch_types=[
            pltpu.VMEM((SUP,), jnp.int32),
            pltpu.VMEM((SUP, P2), jnp.float32),
            pltpu.SemaphoreType.DMA,
        ],
    )
    def k(table_hbm, src_hbm, u_hbm, idx_v, rows_v, sem):
        wid = lax.axis_index("s") * NC + lax.axis_index("c")
        base = wid * PER_W

        def sup_body(g, carry):
            off = base + g * SUP
            pltpu.sync_copy(src_hbm.at[pl.ds(off, SUP)], idx_v)
            copies = []
            for c in range(NCH):
                copies.append(pltpu.async_copy(
                    table_hbm.at[idx_v.at[pl.ds(c * CH, CH)]],
                    rows_v.at[pl.ds(c * CH, CH)], sem))
            for cp in copies:
                cp.wait()
            pltpu.sync_copy(rows_v, u_hbm.at[pl.ds(off, SUP)])
            return carry

        lax.fori_loop(0, NSUP, sup_body, 0)

    return k(table, src)


def _sc_scatter(msg, dst3, zeros):
    mesh = plsc.VectorSubcoreMesh(core_axis_name="c", subcore_axis_name="s")

    @functools.partial(
        pl.kernel,
        out_type=jax.ShapeDtypeStruct((NC, N, P2), jnp.float32),
        mesh=mesh,
        compiler_params=_SC_PARAMS,
        scratch_types=[
            pltpu.VMEM((NSUP * NCH, CH), jnp.int32),
            pltpu.VMEM((SUP, P2), jnp.float32),
            pltpu.SemaphoreType.DMA,
            pltpu.VMEM_SHARED((N, P2), jnp.float32),
        ],
    )
    def k(msg_hbm, dst_hbm, zeros_hbm, out_hbm, idx_v, rows_v, sem, acc):
        cid = lax.axis_index("c")
        sid = lax.axis_index("s")
        wid = sid * NC + cid
        base = wid * PER_W
        pltpu.sync_copy(dst_hbm.at[wid], idx_v)

        @pl.when(sid == 0)
        def _():
            pltpu.sync_copy(zeros_hbm, acc)

        plsc.subcore_barrier()
        for g in range(NSUP):
            pltpu.sync_copy(msg_hbm.at[pl.ds(base + g * SUP, SUP)], rows_v)
            for c in range(NCH):
                pltpu.sync_copy(rows_v.at[pl.ds(c * CH, CH)],
                                acc.at[idx_v.at[g * NCH + c]], add=True)
        plsc.subcore_barrier()
        pltpu.sync_copy(acc.at[pl.ds(sid * ROWS_PER_TILE, ROWS_PER_TILE)],
                        out_hbm.at[cid].at[pl.ds(sid * ROWS_PER_TILE,
                                                 ROWS_PER_TILE)])

    return k(msg, dst3, zeros)


def _sc_counts(dst3, zeros, ones_tile):
    mesh = plsc.VectorSubcoreMesh(core_axis_name="c", subcore_axis_name="s")

    @functools.partial(
        pl.kernel,
        out_type=jax.ShapeDtypeStruct((NC, N, P2), jnp.float32),
        mesh=mesh,
        compiler_params=_SC_PARAMS,
        scratch_types=[
            pltpu.VMEM((NSUP * NCH, CH), jnp.int32),
            pltpu.VMEM((CH, P2), jnp.float32),
            pltpu.VMEM_SHARED((N, P2), jnp.float32),
        ],
    )
    def k(dst_hbm, zeros_hbm, ones_hbm, out_hbm, idx_v, ones_v, acc):
        cid = lax.axis_index("c")
        sid = lax.axis_index("s")
        wid = sid * NC + cid
        pltpu.sync_copy(dst_hbm.at[wid], idx_v)
        pltpu.sync_copy(ones_hbm, ones_v)

        @pl.when(sid == 0)
        def _():
            pltpu.sync_copy(zeros_hbm, acc)

        plsc.subcore_barrier()
        for j in range(NSUP * NCH):
            pltpu.sync_copy(ones_v, acc.at[idx_v.at[j]], add=True)
        plsc.subcore_barrier()
        pltpu.sync_copy(acc.at[pl.ds(sid * ROWS_PER_TILE, ROWS_PER_TILE)],
                        out_hbm.at[cid].at[pl.ds(sid * ROWS_PER_TILE,
                                                 ROWS_PER_TILE)])

    return k(dst3, zeros, ones_tile)


def kernel(x, edge_index, edge_attr, batch, W0, g0, b0, We1, ge1, be1,
           We2, ge2, be2, Wroot, bconv, Wih, Whh, bih, bhh,
           lWih, lWhh, lbih, lbhh, W1, b1, W2, b2, Wf, bf):
    s = 1.0 / jnp.sqrt(1.0 + EPS)
    w0eff = W0 * (g0 * s)[None, :]
    w1eff = We1 * (ge1 * s)[None, :]
    w2eff = We2 * (ge2 * s)[None, :]

    ei = edge_index.astype(jnp.int32)
    src = ei[0]
    dst3 = ei[1].reshape(NW, NSUP * NCH, CH)
    batch2 = batch.astype(jnp.int32).reshape(N, 1)
    zeros = jnp.zeros((N, P2), jnp.float32)
    ones_tile = jnp.ones((CH, P2), jnp.float32)

    h = _tc_prep(x, w0eff, b0.reshape(1, P2))
    cnt2 = _sc_counts(dst3, zeros, ones_tile)

    for _ in range(3):
        u = _sc_gather(h, src)
        msg = _tc_msg(edge_attr, u, w1eff, be1.reshape(1, P1),
                      w2eff, be2.reshape(1, P2 * P2))
        aggr2 = _sc_scatter(msg, dst3, zeros)
        h = _tc_update(h, aggr2, cnt2, Wroot, bconv.reshape(1, P2),
                       Wih, Whh, bih.reshape(1, 3 * P2), bhh.reshape(1, 3 * P2))

    y = _tc_s2s(h, batch2, lWih, lWhh, lbih.reshape(1, 4 * P2),
                lbhh.reshape(1, 4 * P2), W1, b1.reshape(1, P2),
                W2, b2.reshape(1, P2 // 2), Wf, bf.reshape(1, 1))
    return y.reshape(B)


# speedup vs baseline: 3.5761x; 3.5761x over previous
import functools

import jax
import jax.numpy as jnp
from jax import lax
from jax.experimental import pallas as pl
from jax.experimental.pallas import tpu as pltpu
from jax.experimental.pallas import tpu_sc as plsc

N = 10000
E = 320000
DX = 128
DE = 16
P1 = 64
P2 = 32
B = 64
EPS = 1e-5

NC = 2
NS = 16
NW = NC * NS
PER_W = E // NW
NSUP = 5
SUP = PER_W // NSUP
CH = 80
NCH = SUP // CH
ROWS_PER_TILE = N // NS


def _leaky(v):
    return jnp.where(v >= 0, v, 0.01 * v)


_SC_PARAMS = pltpu.CompilerParams(use_tc_tiling_on_sc=False)


def _prep_body(x_ref, w_ref, b_ref, o_ref):
    o_ref[...] = _leaky(
        jnp.dot(x_ref[...], w_ref[...], preferred_element_type=jnp.float32)
        + b_ref[...])


def _tc_prep(x, w0eff, b0):
    return pl.pallas_call(
        _prep_body,
        out_shape=jax.ShapeDtypeStruct((N, P2), jnp.float32),
    )(x, w0eff, b0)


MSG_T = 1280


def _msg_body(ea_ref, u_ref, w1_ref, b1_ref, w2_ref, b2_ref, rep_ref, o_ref):
    ew1 = _leaky(
        jnp.dot(ea_ref[...], w1_ref[...], preferred_element_type=jnp.float32)
        + b1_ref[...])
    wt = jnp.dot(ew1.astype(jnp.bfloat16), w2_ref[...],
                 preferred_element_type=jnp.float32) + b2_ref[...]
    urep = jnp.dot(u_ref[...].astype(jnp.bfloat16), rep_ref[...],
                   preferred_element_type=jnp.float32)
    p = wt * urep
    q = p[:, 0:128]
    for a in range(1, 8):
        q = q + p[:, a * 128:(a + 1) * 128]
    o_ref[...] = (q[:, 0:P2] + q[:, P2:2 * P2]
                  + q[:, 2 * P2:3 * P2] + q[:, 3 * P2:4 * P2])


def _tc_msg(ea, u, w1, b1, w2, b2, rep):
    grid = (E // MSG_T,)
    return pl.pallas_call(
        _msg_body,
        grid=grid,
        in_specs=[
            pl.BlockSpec((MSG_T, DE), lambda i: (i, 0)),
            pl.BlockSpec((MSG_T, P2), lambda i: (i, 0)),
            pl.BlockSpec((DE, P1), lambda i: (0, 0)),
            pl.BlockSpec((1, P1), lambda i: (0, 0)),
            pl.BlockSpec((P1, P2 * P2), lambda i: (0, 0)),
            pl.BlockSpec((1, P2 * P2), lambda i: (0, 0)),
            pl.BlockSpec((P2, P2 * P2), lambda i: (0, 0)),
        ],
        out_specs=pl.BlockSpec((MSG_T, P2), lambda i: (i, 0)),
        out_shape=jax.ShapeDtypeStruct((E, P2), jnp.float32),
        compiler_params=pltpu.CompilerParams(
            dimension_semantics=("arbitrary",)),
    )(ea, u, w1, b1, w2, b2, rep)


def _update_body(h_ref, ag_ref, ct_ref, wr_ref, bc_ref, wih_ref, whh_ref,
                 bih_ref, bhh_ref, o_ref):
    h = h_ref[...]
    aggr = (ag_ref[0] + ag_ref[1]) / jnp.maximum(ct_ref[0] + ct_ref[1], 1.0)
    m = _leaky(
        jnp.dot(h, wr_ref[...], preferred_element_type=jnp.float32)
        + aggr + bc_ref[...])
    gi = jnp.dot(m, wih_ref[...], preferred_element_type=jnp.float32) + bih_ref[...]
    gh = jnp.dot(h, whh_ref[...], preferred_element_type=jnp.float32) + bhh_ref[...]
    r = jax.nn.sigmoid(gi[:, 0:P2] + gh[:, 0:P2])
    z = jax.nn.sigmoid(gi[:, P2:2 * P2] + gh[:, P2:2 * P2])
    n = jnp.tanh(gi[:, 2 * P2:3 * P2] + r * gh[:, 2 * P2:3 * P2])
    o_ref[...] = (1.0 - z) * n + z * h


def _tc_update(h, aggr2, cnt2, wroot, bconv, wih, whh, bih, bhh):
    return pl.pallas_call(
        _update_body,
        out_shape=jax.ShapeDtypeStruct((N, P2), jnp.float32),
    )(h, aggr2, cnt2, wroot, bconv, wih, whh, bih, bhh)


def _s2s_body(x_ref, b_ref, lwih_ref, lwhh_ref, lbih_ref, lbhh_ref,
              w1_ref, b1_ref, w2_ref, b2_ref, wf_ref, bf_ref, y_ref):
    xo = x_ref[...]
    mask = b_ref[...] == lax.broadcasted_iota(jnp.int32, (N, B), 1)
    q_star = jnp.zeros((B, 2 * P2), jnp.float32)
    hl = jnp.zeros((B, P2), jnp.float32)
    cl = jnp.zeros((B, P2), jnp.float32)
    for _ in range(3):
        gates = (jnp.dot(q_star, lwih_ref[...], preferred_element_type=jnp.float32)
                 + lbih_ref[...]
                 + jnp.dot(hl, lwhh_ref[...], preferred_element_type=jnp.float32)
                 + lbhh_ref[...])
        i_ = jax.nn.sigmoid(gates[:, 0:P2])
        f_ = jax.nn.sigmoid(gates[:, P2:2 * P2])
        g_ = jnp.tanh(gates[:, 2 * P2:3 * P2])
        o_ = jax.nn.sigmoid(gates[:, 3 * P2:4 * P2])
        cl = f_ * cl + i_ * g_
        hl = o_ * jnp.tanh(cl)
        em = lax.dot_general(xo, hl, (((1,), (1,)), ((), ())),
                             preferred_element_type=jnp.float32)
        e = jnp.sum(jnp.where(mask, em, 0.0), axis=1, keepdims=True)
        emax = jnp.max(jnp.where(mask, e, -1e30), axis=0, keepdims=True)
        esel = jnp.sum(jnp.where(mask, emax, 0.0), axis=1, keepdims=True)
        a_un = jnp.exp(e - esel)
        masked_a = jnp.where(mask, a_un, 0.0)
        asum = jnp.maximum(jnp.sum(masked_a, axis=0, keepdims=True), 1e-30)
        wa = masked_a / asum
        rvec = lax.dot_general(wa, xo, (((0,), (0,)), ((), ())),
                               preferred_element_type=jnp.float32)
        q_star = jnp.concatenate([hl, rvec], axis=1)
    y = _leaky(jnp.dot(q_star, w1_ref[...], preferred_element_type=jnp.float32)
               + b1_ref[...])
    y = _leaky(jnp.dot(y, w2_ref[...], preferred_element_type=jnp.float32)
               + b2_ref[...])
    y_ref[...] = jnp.dot(y, wf_ref[...], preferred_element_type=jnp.float32) \
        + bf_ref[...]


def _tc_s2s(xo, batch2, lwih, lwhh, lbih, lbhh, w1, b1, w2, b2, wf, bf):
    return pl.pallas_call(
        _s2s_body,
        out_shape=jax.ShapeDtypeStruct((B, 1), jnp.float32),
    )(xo, batch2, lwih, lwhh, lbih, lbhh, w1, b1, w2, b2, wf, bf)


def _sc_gather(table, src):
    mesh = plsc.VectorSubcoreMesh(core_axis_name="c", subcore_axis_name="s")

    @functools.partial(
        pl.kernel,
        out_type=jax.ShapeDtypeStruct((E, P2), jnp.float32),
        mesh=mesh,
        compiler_params=_SC_PARAMS,
        scratch_types=[
            pltpu.VMEM((SUP,), jnp.int32),
            pltpu.VMEM((SUP, P2), jnp.float32),
            pltpu.SemaphoreType.DMA,
        ],
    )
    def k(table_hbm, src_hbm, u_hbm, idx_v, rows_v, sem):
        wid = lax.axis_index("s") * NC + lax.axis_index("c")
        base = wid * PER_W

        def sup_body(g, carry):
            off = base + g * SUP
            pltpu.sync_copy(src_hbm.at[pl.ds(off, SUP)], idx_v)
            copies = []
            for c in range(NCH):
                copies.append(pltpu.async_copy(
                    table_hbm.at[idx_v.at[pl.ds(c * CH, CH)]],
                    rows_v.at[pl.ds(c * CH, CH)], sem))
            for cp in copies:
                cp.wait()
            pltpu.sync_copy(rows_v, u_hbm.at[pl.ds(off, SUP)])
            return carry

        lax.fori_loop(0, NSUP, sup_body, 0)

    return k(table, src)


def _sc_scatter(msg, dst3, zeros):
    mesh = plsc.VectorSubcoreMesh(core_axis_name="c", subcore_axis_name="s")

    @functools.partial(
        pl.kernel,
        out_type=jax.ShapeDtypeStruct((NC, N, P2), jnp.float32),
        mesh=mesh,
        compiler_params=_SC_PARAMS,
        scratch_types=[
            pltpu.VMEM((NSUP * NCH, CH), jnp.int32),
            pltpu.VMEM((SUP, P2), jnp.float32),
            pltpu.SemaphoreType.DMA,
            pltpu.VMEM_SHARED((N, P2), jnp.float32),
        ],
    )
    def k(msg_hbm, dst_hbm, zeros_hbm, out_hbm, idx_v, rows_v, sem, acc):
        cid = lax.axis_index("c")
        sid = lax.axis_index("s")
        wid = sid * NC + cid
        base = wid * PER_W
        pltpu.sync_copy(dst_hbm.at[wid], idx_v)

        @pl.when(sid == 0)
        def _():
            pltpu.sync_copy(zeros_hbm, acc)

        plsc.subcore_barrier()
        for g in range(NSUP):
            pltpu.sync_copy(msg_hbm.at[pl.ds(base + g * SUP, SUP)], rows_v)
            for c in range(NCH):
                pltpu.sync_copy(rows_v.at[pl.ds(c * CH, CH)],
                                acc.at[idx_v.at[g * NCH + c]], add=True)
        plsc.subcore_barrier()
        pltpu.sync_copy(acc.at[pl.ds(sid * ROWS_PER_TILE, ROWS_PER_TILE)],
                        out_hbm.at[cid].at[pl.ds(sid * ROWS_PER_TILE,
                                                 ROWS_PER_TILE)])

    return k(msg, dst3, zeros)


def _sc_counts(dst3, zeros, ones_tile):
    mesh = plsc.VectorSubcoreMesh(core_axis_name="c", subcore_axis_name="s")

    @functools.partial(
        pl.kernel,
        out_type=jax.ShapeDtypeStruct((NC, N, P2), jnp.float32),
        mesh=mesh,
        compiler_params=_SC_PARAMS,
        scratch_types=[
            pltpu.VMEM((NSUP * NCH, CH), jnp.int32),
            pltpu.VMEM((CH, P2), jnp.float32),
            pltpu.VMEM_SHARED((N, P2), jnp.float32),
        ],
    )
    def k(dst_hbm, zeros_hbm, ones_hbm, out_hbm, idx_v, ones_v, acc):
        cid = lax.axis_index("c")
        sid = lax.axis_index("s")
        wid = sid * NC + cid
        pltpu.sync_copy(dst_hbm.at[wid], idx_v)
        pltpu.sync_copy(ones_hbm, ones_v)

        @pl.when(sid == 0)
        def _():
            pltpu.sync_copy(zeros_hbm, acc)

        plsc.subcore_barrier()
        for j in range(NSUP * NCH):
            pltpu.sync_copy(ones_v, acc.at[idx_v.at[j]], add=True)
        plsc.subcore_barrier()
        pltpu.sync_copy(acc.at[pl.ds(sid * ROWS_PER_TILE, ROWS_PER_TILE)],
                        out_hbm.at[cid].at[pl.ds(sid * ROWS_PER_TILE,
                                                 ROWS_PER_TILE)])

    return k(dst3, zeros, ones_tile)


def kernel(x, edge_index, edge_attr, batch, W0, g0, b0, We1, ge1, be1,
           We2, ge2, be2, Wroot, bconv, Wih, Whh, bih, bhh,
           lWih, lWhh, lbih, lbhh, W1, b1, W2, b2, Wf, bf):
    s = 1.0 / jnp.sqrt(1.0 + EPS)
    w0eff = W0 * (g0 * s)[None, :]
    w1eff = We1 * (ge1 * s)[None, :]
    w2eff = We2 * (ge2 * s)[None, :]

    ei = edge_index.astype(jnp.int32)
    src = ei[0]
    dst3 = ei[1].reshape(NW, NSUP * NCH, CH)
    batch2 = batch.astype(jnp.int32).reshape(N, 1)
    zeros = jnp.zeros((N, P2), jnp.float32)
    ones_tile = jnp.ones((CH, P2), jnp.float32)
    rep = jnp.kron(jnp.eye(P2, dtype=jnp.bfloat16),
                   jnp.ones((1, P2), jnp.bfloat16))
    w2bf = w2eff.astype(jnp.bfloat16)

    h = _tc_prep(x, w0eff, b0.reshape(1, P2))
    cnt2 = _sc_counts(dst3, zeros, ones_tile)

    for _ in range(3):
        u = _sc_gather(h, src)
        msg = _tc_msg(edge_attr, u, w1eff, be1.reshape(1, P1),
                      w2bf, be2.reshape(1, P2 * P2), rep)
        aggr2 = _sc_scatter(msg, dst3, zeros)
        h = _tc_update(h, aggr2, cnt2, Wroot, bconv.reshape(1, P2),
                       Wih, Whh, bih.reshape(1, 3 * P2), bhh.reshape(1, 3 * P2))

    y = _tc_s2s(h, batch2, lWih, lWhh, lbih.reshape(1, 4 * P2),
                lbhh.reshape(1, 4 * P2), W1, b1.reshape(1, P2),
                W2, b2.reshape(1, P2 // 2), Wf, bf.reshape(1, 1))
    return y.reshape(B)
